# baseline (speedup 1.0000x reference)
import functools

import jax
import jax.numpy as jnp
from jax import lax
from jax.experimental import pallas as pl
from jax.experimental.pallas import tpu as pltpu
from jax.experimental.pallas import tpu_sc as plsc

_B, _S = 8, 32
_NCH, _NFEAT, _D = 4, 32, 2048
_NLEV = 100
_R = _B * _S * _NCH

_NC, _NS, _L = 2, 16, 16
_NW = _NC * _NS
_W = _D // _NW
_KV = _W // _L


def _quant_body(samples_ref, idx_ref):
    s = samples_ref[...]
    idxf = jnp.round((s + 1.0) / 2.0 * 99.0)
    idx_ref[...] = jnp.clip(idxf, 0.0, 99.0).astype(jnp.int32)


@functools.partial(
    pl.kernel,
    mesh=plsc.VectorSubcoreMesh(core_axis_name="c", subcore_axis_name="s"),
    out_type=jax.ShapeDtypeStruct((_NW, _R * _W), jnp.float32),
    scratch_types=[
        pltpu.VMEM((_R * _NFEAT,), jnp.int32),
        pltpu.VMEM((_NLEV * _W,), jnp.float32),
        pltpu.VMEM((_NFEAT * _W,), jnp.float32),
        pltpu.VMEM((_NCH * _W,), jnp.float32),
        pltpu.VMEM((_R * _W,), jnp.float32),
    ],
)
def _sc_s4(idx_hbm, vw_hbm, fw_hbm, cw_hbm, s4_hbm,
           idx_v, vw_v, fw_v, cw_v, s4_v):
    wid = lax.axis_index("s") * _NC + lax.axis_index("c")
    pltpu.sync_copy(idx_hbm, idx_v)
    pltpu.sync_copy(vw_hbm.at[wid], vw_v)
    pltpu.sync_copy(fw_hbm.at[wid], fw_v)
    pltpu.sync_copy(cw_hbm.at[wid], cw_v)

    csum = [cw_v[pl.ds(0 * _W + k * _L, _L)] + cw_v[pl.ds(1 * _W + k * _L, _L)]
            + cw_v[pl.ds(2 * _W + k * _L, _L)] + cw_v[pl.ds(3 * _W + k * _L, _L)]
            for k in range(_KV)]

    def row_body(r, _):
        ivecs = [idx_v[pl.ds(r * _NFEAT + j * _L, _L)]
                 for j in range(_NFEAT // _L)]
        accs = [jnp.zeros((_L,), jnp.float32) for _ in range(_KV)]
        for f in range(_NFEAT):
            i = ivecs[f // _L][f % _L]
            base = i * _W
            for k in range(_KV):
                accs[k] = accs[k] + (vw_v[pl.ds(base + k * _L, _L)]
                                     * fw_v[pl.ds(f * _W + k * _L, _L)])
        for k in range(_KV):
            s4_v[pl.ds(r * _W + k * _L, _L)] = accs[k] * csum[k]
        return 0

    lax.fori_loop(0, _R, row_body, 0)
    pltpu.sync_copy(s4_v, s4_hbm.at[wid])


def _to_worker_major(x):
    n = x.shape[0]
    return x.reshape(n, _NW, _W).transpose(1, 0, 2).reshape(_NW, n * _W)


def kernel(samples, component_weight, feat_weight, value_weight):
    samples_r = samples.reshape(_R, _NFEAT)
    idx = pl.pallas_call(
        _quant_body,
        out_shape=jax.ShapeDtypeStruct((_R, _NFEAT), jnp.int32),
    )(samples_r)

    s4w = _sc_s4(idx.reshape(-1),
                 _to_worker_major(value_weight),
                 _to_worker_major(feat_weight),
                 _to_worker_major(component_weight))
    s4 = (s4w.reshape(_NW, _R, _W).transpose(1, 0, 2)
          .reshape(_B, _S, _NCH, _D))

    t_interp = jax.nn.sigmoid(s4[:, :, 2, :] + s4[:, :, 3, :])
    h = s4[:, :, 0, :] * (1.0 - t_interp) + t_interp * s4[:, :, 1, :]
    h = jnp.roll(h, shift=1, axis=0)
    h = h.at[0].set(jnp.zeros_like(h[0]))
    s4 = s4 + h[:, :, None, :]
    return jnp.sign(jnp.sum(s4.reshape(_B, _S, -1), axis=1))


# speedup vs baseline: 1.6045x; 1.6045x over previous
import functools

import jax
import jax.numpy as jnp
from jax import lax
from jax.experimental import pallas as pl
from jax.experimental.pallas import tpu as pltpu
from jax.experimental.pallas import tpu_sc as plsc

_B, _S = 8, 32
_NCH, _NFEAT, _D = 4, 32, 2048
_NLEV = 100
_R = _B * _S * _NCH

_NC, _NS, _L = 2, 16, 16
_NW = _NC * _NS
_W = _D // _NW
_NQ = _W // _L
_FG = 8
_NG = _NFEAT // _FG


def _prep_body(samples_ref, vw_ref, fw_ref, cw_ref,
               av_ref, tab_ref, off_ref):
    s = samples_ref[...]
    idxf = jnp.round((s + 1.0) / 2.0 * 99.0)
    idx = jnp.clip(idxf, 0.0, 99.0).astype(jnp.int32)
    av_ref[...] = idx * _W

    csum = jnp.sum(cw_ref[...], axis=0, keepdims=True)
    vwc = vw_ref[...] * csum
    tab_ref[0:_NLEV, :] = vwc
    tab_ref[_NLEV:2 * _NLEV, :] = -vwc
    tab_ref[2 * _NLEV:3 * _NLEV, :] = jnp.zeros_like(vwc)

    fw = fw_ref[...]
    dloc = lax.broadcasted_iota(jnp.int32, (_NFEAT, _D), 1) % _W
    off = jnp.where(fw < 0.0, _NLEV * _W, 0)
    off = jnp.where(fw == 0.0, 2 * _NLEV * _W, off)
    off_ref[...] = off + dloc


@functools.partial(
    pl.kernel,
    mesh=plsc.VectorSubcoreMesh(core_axis_name="c", subcore_axis_name="s"),
    out_type=jax.ShapeDtypeStruct((_NW, _R * _W), jnp.float32),
    scratch_types=[
        pltpu.VMEM((_NFEAT * _R,), jnp.int32),
        pltpu.VMEM((3 * _NLEV * _W,), jnp.float32),
        pltpu.VMEM((_NFEAT * _W,), jnp.int32),
        pltpu.VMEM((_R * _W,), jnp.float32),
    ],
    compiler_params=pltpu.CompilerParams(needs_layout_passes=False),
)
def _sc_s4(av_hbm, tab_hbm, off_hbm, s4_hbm, av_v, tab_v, off_v, s4_v):
    wid = lax.axis_index("s") * _NC + lax.axis_index("c")
    pltpu.sync_copy(av_hbm, av_v)
    pltpu.sync_copy(tab_hbm.at[wid], tab_v)
    pltpu.sync_copy(off_hbm.at[wid], off_v)

    for g in range(_NG):
        offs = [[off_v[pl.ds((g * _FG + f) * _W + q * _L, _L)]
                 for q in range(_NQ)] for f in range(_FG)]

        def r_body(r, _, g=g, offs=offs):
            if g == 0:
                accs = [jnp.zeros((_L,), jnp.float32) for _ in range(_NQ)]
            else:
                accs = [s4_v[pl.ds(r * _W + q * _L, _L)] for q in range(_NQ)]
            for f in range(_FG):
                ai = plsc.load_gather(
                    av_v, [jnp.full((_L,), (g * _FG + f) * _R + r,
                                    dtype=jnp.int32)])
                for q in range(_NQ):
                    accs[q] = accs[q] + plsc.load_gather(
                        tab_v, [ai + offs[f][q]])
            for q in range(_NQ):
                s4_v[pl.ds(r * _W + q * _L, _L)] = accs[q]
            return 0

        lax.fori_loop(0, _R, r_body, 0)

    pltpu.sync_copy(s4_v, s4_hbm.at[wid])


def kernel(samples, component_weight, feat_weight, value_weight):
    samples_r = samples.reshape(_R, _NFEAT)
    av, tab, off = pl.pallas_call(
        _prep_body,
        out_shape=(
            jax.ShapeDtypeStruct((_R, _NFEAT), jnp.int32),
            jax.ShapeDtypeStruct((3 * _NLEV, _D), jnp.float32),
            jax.ShapeDtypeStruct((_NFEAT, _D), jnp.int32),
        ),
    )(samples_r, value_weight, feat_weight, component_weight)

    av_t = av.T.reshape(-1)
    tab_wm = (tab.reshape(3 * _NLEV, _NW, _W)
              .transpose(1, 0, 2).reshape(_NW, 3 * _NLEV * _W))
    off_wm = (off.reshape(_NFEAT, _NW, _W)
              .transpose(1, 0, 2).reshape(_NW, _NFEAT * _W))

    s4w = _sc_s4(av_t, tab_wm, off_wm)
    s4 = (s4w.reshape(_NW, _R, _W).transpose(1, 0, 2)
          .reshape(_B, _S, _NCH, _D))

    t_interp = jax.nn.sigmoid(s4[:, :, 2, :] + s4[:, :, 3, :])
    h = s4[:, :, 0, :] * (1.0 - t_interp) + t_interp * s4[:, :, 1, :]
    h = jnp.roll(h, shift=1, axis=0)
    h = h.at[0].set(jnp.zeros_like(h[0]))
    s4 = s4 + h[:, :, None, :]
    return jnp.sign(jnp.sum(s4.reshape(_B, _S, -1), axis=1))


# speedup vs baseline: 1.8598x; 1.1592x over previous
import functools

import jax
import jax.numpy as jnp
from jax import lax
from jax.experimental import pallas as pl
from jax.experimental.pallas import tpu as pltpu
from jax.experimental.pallas import tpu_sc as plsc

_B, _S = 8, 32
_NCH, _NFEAT, _D = 4, 32, 2048
_NLEV = 100
_R = _B * _S * _NCH

_NC, _NS, _L = 2, 16, 16
_NW = _NC * _NS
_WC = 128
_WR = _R // 2
_NQ = _WC // _L
_FG = 4
_NG = _NFEAT // _FG


def _prep_body(samples_ref, vw_ref, fw_ref, cw_ref,
               av_ref, vwc_ref, off_ref):
    s = samples_ref[...]
    idxf = jnp.round((s + 1.0) / 2.0 * 99.0)
    av_ref[...] = jnp.clip(idxf, 0.0, 99.0).astype(jnp.int32)

    csum = jnp.sum(cw_ref[...], axis=0, keepdims=True)
    vwc_ref[...] = vw_ref[...] * csum

    fw = fw_ref[...]
    off = jnp.where(fw < 0.0, _NLEV, 0)
    off_ref[...] = jnp.where(fw == 0.0, 2 * _NLEV, off)


@functools.partial(
    pl.kernel,
    mesh=plsc.VectorSubcoreMesh(core_axis_name="c", subcore_axis_name="s"),
    out_type=jax.ShapeDtypeStruct((_R, _D), jnp.float32),
    scratch_types=[
        pltpu.VMEM((_WR * _NFEAT,), jnp.int32),
        pltpu.VMEM((3 * _NLEV, _WC), jnp.float32),
        pltpu.VMEM((_NFEAT, _WC), jnp.int32),
        pltpu.VMEM((_WR, _WC), jnp.float32),
    ],
    compiler_params=pltpu.CompilerParams(needs_layout_passes=False),
)
def _sc_s4(av_hbm, vwc_hbm, off_hbm, s4_hbm, av_v, tab_v, off_v, s4_v):
    wid = lax.axis_index("s") * _NC + lax.axis_index("c")
    ctile = wid // 2
    rhalf = wid % 2
    cols = pl.ds(ctile * _WC, _WC)
    pltpu.sync_copy(av_hbm.at[pl.ds(rhalf * _WR * _NFEAT, _WR * _NFEAT)],
                    av_v)
    pltpu.sync_copy(vwc_hbm.at[:, cols], tab_v.at[pl.ds(0, _NLEV)])
    pltpu.sync_copy(off_hbm.at[:, cols], off_v)

    zero = jnp.zeros((_L,), jnp.float32)

    def neg_body(i, _):
        for q in range(_NQ):
            v = tab_v[i, pl.ds(q * _L, _L)]
            tab_v[_NLEV + i, pl.ds(q * _L, _L)] = -v
            tab_v[2 * _NLEV + i, pl.ds(q * _L, _L)] = zero
        return 0

    lax.fori_loop(0, _NLEV, neg_body, 0)

    colqs = [lax.iota(jnp.int32, _L) + q * _L for q in range(_NQ)]

    for g in range(_NG):
        offs = [[off_v[g * _FG + f, pl.ds(q * _L, _L)] for q in range(_NQ)]
                for f in range(_FG)]

        def r_body(r, _, g=g, offs=offs):
            if g == 0:
                accs = [jnp.zeros((_L,), jnp.float32) for _ in range(_NQ)]
            else:
                accs = [s4_v[r, pl.ds(q * _L, _L)] for q in range(_NQ)]
            for f in range(_FG):
                ai = plsc.load_gather(
                    av_v, [jnp.full((_L,), r * _NFEAT + g * _FG + f,
                                    dtype=jnp.int32)])
                for q in range(_NQ):
                    accs[q] = accs[q] + plsc.load_gather(
                        tab_v, [ai + offs[f][q], colqs[q]])
            for q in range(_NQ):
                s4_v[r, pl.ds(q * _L, _L)] = accs[q]
            return 0

        lax.fori_loop(0, _WR, r_body, 0)

    pltpu.sync_copy(s4_v, s4_hbm.at[pl.ds(rhalf * _WR, _WR), cols])


def kernel(samples, component_weight, feat_weight, value_weight):
    samples_r = samples.reshape(_R, _NFEAT)
    av, vwc, off = pl.pallas_call(
        _prep_body,
        out_shape=(
            jax.ShapeDtypeStruct((_R, _NFEAT), jnp.int32),
            jax.ShapeDtypeStruct((_NLEV, _D), jnp.float32),
            jax.ShapeDtypeStruct((_NFEAT, _D), jnp.int32),
        ),
    )(samples_r, value_weight, feat_weight, component_weight)

    s4 = _sc_s4(av.reshape(-1), vwc, off).reshape(_B, _S, _NCH, _D)

    t_interp = jax.nn.sigmoid(s4[:, :, 2, :] + s4[:, :, 3, :])
    h = s4[:, :, 0, :] * (1.0 - t_interp) + t_interp * s4[:, :, 1, :]
    h = jnp.roll(h, shift=1, axis=0)
    h = h.at[0].set(jnp.zeros_like(h[0]))
    s4 = s4 + h[:, :, None, :]
    return jnp.sign(jnp.sum(s4.reshape(_B, _S, -1), axis=1))
